# baseline (speedup 1.0000x reference)
import functools

import jax
import jax.numpy as jnp
from jax import lax
from jax.experimental import pallas as pl
from jax.experimental.pallas import tpu as pltpu
from jax.experimental.pallas import tpu_sc as plsc

D = 64
L = 16
BBLK = 128
NBUF = 3


def _rsqrt(x):
    i = plsc.bitcast(x, jnp.int32)
    i = jnp.int32(0x5F3759DF) - (i >> 1)
    y = plsc.bitcast(i, jnp.float32)
    h = x * jnp.float32(0.5)
    for _ in range(2):
        y = y * (jnp.float32(1.5) - h * y * y)
    return y


def _rot_bases(lanes):
    return [(lanes + m) & (L - 1) for m in range(L)]


def _pack_table(tt, tail_tt, rows):
    info = plsc.get_sparse_core_info()
    nc, ns = info.num_cores, info.num_subcores
    nw = nc * ns
    nblk_full = rows // BBLK
    per_w = nblk_full // nw
    extra = nblk_full - per_w * nw
    mesh = plsc.VectorSubcoreMesh(core_axis_name="c", subcore_axis_name="s")

    @functools.partial(
        pl.kernel,
        mesh=mesh,
        out_type=jax.ShapeDtypeStruct((rows, 2 * D), jnp.float32),
        compiler_params=pltpu.CompilerParams(needs_layout_passes=False),
        scratch_types=[
            pltpu.VMEM((NBUF, D, BBLK), jnp.float32),
            pltpu.VMEM((NBUF, BBLK, 2 * D), jnp.float32),
            pltpu.SemaphoreType.DMA((NBUF,)),
            pltpu.SemaphoreType.DMA((NBUF,)),
        ],
    )
    def body(tt_hbm, tail_hbm, out_hbm, tin_v, tob_v, sem_i, sem_o):
        wid = lax.axis_index("s") * nc + lax.axis_index("c")
        base = wid * per_w

        def col0(i):
            main = (base + i) * BBLK
            left = (per_w * nw + wid) * BBLK
            ex = jnp.where(wid < extra, left, base * BBLK)
            return pl.multiple_of(jnp.where(i < per_w, main, ex), BBLK)

        def slot(i):
            return lax.rem(i, NBUF)

        def in_dma(i):
            b = slot(i)
            return pltpu.make_async_copy(
                tt_hbm.at[:, pl.ds(col0(i), BBLK)], tin_v.at[b], sem_i.at[b])

        def out_dma(i):
            b = slot(i)
            return pltpu.make_async_copy(
                tob_v.at[b], out_hbm.at[pl.ds(col0(i), BBLK)], sem_o.at[b])

        in_dma(0).start()
        in_dma(1).start()

        lanes = lax.iota(jnp.int32, L)
        rb = _rot_bases(lanes)
        n_slots = per_w + 1

        def blk(i, carry):
            b = slot(i)

            @pl.when(i + 2 < n_slots)
            def _():
                in_dma(i + 2).start()

            in_dma(i).wait()

            @pl.when(i >= NBUF)
            def _():
                out_dma(i - NBUF).wait()

            tin = tin_v.at[b]
            tob = tob_v.at[b]

            def grp(t, c):
                rowv = t * L + lanes
                for d in range(D):
                    rv = rb[d & (L - 1)] + (d & ~(L - 1))
                    v = plsc.load_gather(tin, [rv, rowv])
                    plsc.store_scatter(tob, [rowv, rv], v)
                return c

            lax.fori_loop(0, BBLK // L, grp, 0)
            out_dma(i).start()
            return carry

        lax.fori_loop(0, n_slots, blk, 0)
        for t in range(NBUF):
            out_dma(n_slots - 1 - t).wait()

        if rows % BBLK:
            @pl.when(wid == extra)
            def _():
                pltpu.sync_copy(tail_hbm, tin_v.at[0])

                def tgrp(t, c):
                    rowv = t * L + lanes
                    for d in range(D):
                        rv = rb[d & (L - 1)] + (d & ~(L - 1))
                        v = plsc.load_gather(tin_v.at[0], [rv, rowv])
                        plsc.store_scatter(tob_v.at[0], [rowv, rv], v)
                    return c

                lax.fori_loop(0, BBLK // L, tgrp, 0)
                pltpu.sync_copy(tob_v.at[0],
                                out_hbm.at[pl.ds(rows - BBLK, BBLK)])

    return body(tt, tail_tt)


def _lookup_normalize(xt, table_p, hist, batch):
    info = plsc.get_sparse_core_info()
    nc, ns = info.num_cores, info.num_subcores
    mesh = plsc.VectorSubcoreMesh(core_axis_name="c", subcore_axis_name="s")

    @functools.partial(
        pl.kernel,
        mesh=mesh,
        out_type=jax.ShapeDtypeStruct((hist, D, batch), jnp.float32),
        compiler_params=pltpu.CompilerParams(needs_layout_passes=False),
        scratch_types=[
            pltpu.VMEM((hist, BBLK), jnp.int32),
            pltpu.VMEM((NBUF, BBLK, 2 * D), jnp.float32),
            pltpu.VMEM((NBUF, D, BBLK), jnp.float32),
            pltpu.SemaphoreType.DMA((NBUF,)),
            pltpu.SemaphoreType.DMA((NBUF,)),
        ],
    )
    def body(x_hbm, table_hbm, out_hbm, xb_v, gb_v, ob_v, sem_g, sem_o):
        wid = lax.axis_index("s") * nc + lax.axis_index("c")
        bcol = wid * BBLK

        def slot(g):
            return lax.rem(g, NBUF)

        def gather_dma(g):
            b = slot(g)
            return pltpu.make_async_copy(
                table_hbm.at[xb_v.at[g]], gb_v.at[b], sem_g.at[b])

        def out_dma(g):
            b = slot(g)
            return pltpu.make_async_copy(
                ob_v.at[b],
                out_hbm.at[g, :, pl.ds(bcol, BBLK)], sem_o.at[b])

        pltpu.sync_copy(x_hbm.at[:, pl.ds(bcol, BBLK)], xb_v)
        gather_dma(0).start()

        lanes = lax.iota(jnp.int32, L)
        rb = _rot_bases(lanes)

        def chunk_body(g, carry):
            b = slot(g)

            @pl.when(g + 1 < hist)
            def _():
                gather_dma(g + 1).start()

            gather_dma(g).wait()

            @pl.when(g >= NBUF)
            def _():
                out_dma(g - NBUF).wait()

            gb = gb_v.at[b]
            ob = ob_v.at[b]

            def grp(t, c):
                rowv = t * L + lanes
                ss = jnp.zeros((L,), jnp.float32)
                for d in range(D):
                    rv = rb[d & (L - 1)] + (d & ~(L - 1))
                    v = plsc.load_gather(gb, [rowv, rv])
                    ss = ss + v * v
                scale = _rsqrt(ss)
                for d in range(D):
                    rv = rb[d & (L - 1)] + (d & ~(L - 1))
                    v = plsc.load_gather(gb, [rowv, rv])
                    plsc.store_scatter(ob, [rv, rowv], v * scale)
                return c

            lax.fori_loop(0, BBLK // L, grp, 0)
            out_dma(g).start()
            return carry

        lax.fori_loop(0, hist, chunk_body, 0)
        for t in range(NBUF):
            out_dma(hist - 1 - t).wait()

    return body(xt, table_p)


@functools.partial(jax.jit, static_argnames=("hist", "batch", "rows"))
def _embed_normalize(xt, tt, tail_tt, hist, batch, rows):
    table_p = _pack_table(tt, tail_tt, rows)
    return _lookup_normalize(xt, table_p, hist, batch)


def kernel(x, table):
    b, h = x.shape
    out = _embed_normalize(x.T, table.T, table[-BBLK:].T, h, b,
                           table.shape[0])
    return out.transpose(2, 0, 1)
